# baseline (speedup 1.0000x reference)
import jax
import jax.numpy as jnp
from jax.experimental import pallas as pl
from jax.experimental.pallas import tpu as pltpu


def _vq_batch_kernel(z_ref, cbt_ref, zq_ref, idx_ref, loss_ref):
    bg = z_ref.shape[0]
    n_sub = cbt_ref.shape[0]
    ds = cbt_ref.shape[1]
    hw = z_ref.shape[2]
    loss = None
    for g in range(bg):
        idx_rows = []
        for n in range(n_sub):
            z_blk = z_ref[g, n * ds:(n + 1) * ds]
            cbt = cbt_ref[n]

            c2_row = jnp.sum(cbt * cbt, axis=0, keepdims=True)
            c2 = c2_row.T
            ncross2 = jax.lax.dot_general(
                cbt * -2.0, z_blk, (((0,), (0,)), ((), ())),
                preferred_element_type=jnp.float32)
            d2 = ncross2 + c2

            idx = jnp.argmin(d2, axis=0, keepdims=True)

            onehot = (jax.lax.broadcasted_iota(jnp.int32, d2.shape, 0) == idx
                      ).astype(jnp.float32)
            z_q = jax.lax.dot_general(
                cbt, onehot, (((1,), (0,)), ((), ())),
                preferred_element_type=jnp.float32)

            diff = z_q - z_blk
            zq_ref[g, n * ds:(n + 1) * ds] = z_blk + diff
            part = jnp.sum(diff * diff)
            loss = part if loss is None else loss + part
            idx_rows.append(idx.astype(jnp.int32))

        idx_mat = jnp.concatenate(idx_rows, axis=0)
        idx_ref[g * hw:(g + 1) * hw, :] = idx_mat.T
    loss_ref[0, 0, 0] = loss


def kernel(z, codebooks):
    B, D, H, W = z.shape
    N, M, ds = codebooks.shape
    HW = H * W
    T = B * HW

    zr = z.reshape(B, D, HW)
    cbt = codebooks.transpose(0, 2, 1)

    BG = 4
    zq3, idx2, loss_p = pl.pallas_call(
        _vq_batch_kernel,
        grid=(B // BG,),
        in_specs=[
            pl.BlockSpec((BG, D, HW), lambda b: (b, 0, 0)),
            pl.BlockSpec((N, ds, M), lambda b: (0, 0, 0)),
        ],
        out_specs=[
            pl.BlockSpec((BG, D, HW), lambda b: (b, 0, 0)),
            pl.BlockSpec((BG * HW, N), lambda b: (b, 0)),
            pl.BlockSpec((1, 1, 1), lambda b: (b, 0, 0),
                         memory_space=pltpu.SMEM),
        ],
        out_shape=[
            jax.ShapeDtypeStruct((B, D, HW), jnp.float32),
            jax.ShapeDtypeStruct((T, N), jnp.int32),
            jax.ShapeDtypeStruct((B // BG, 1, 1), jnp.float32),
        ],
        compiler_params=pltpu.CompilerParams(
            dimension_semantics=("parallel",)),
    )(zr, cbt)

    z_q_out = zq3.reshape(B, D, H, W)
    loss = jnp.sum(loss_p) / jnp.float32(N * T * ds)
    return (z_q_out, loss, loss, idx2)


# speedup vs baseline: 1.1881x; 1.1881x over previous
import jax
import jax.numpy as jnp
from jax.experimental import pallas as pl
from jax.experimental.pallas import tpu as pltpu


def _vq_batch_kernel(z_ref, cbt_ref, zq_ref, idx_ref, loss_ref):
    bg = z_ref.shape[0]
    n_sub = cbt_ref.shape[0]
    ds = cbt_ref.shape[1]
    hw = z_ref.shape[2]
    loss = None
    for g in range(bg):
        idx_rows = []
        for n in range(n_sub):
            z_blk = z_ref[g, n * ds:(n + 1) * ds]
            cbt = cbt_ref[n]

            c2_row = jnp.sum(cbt * cbt, axis=0, keepdims=True)
            c2 = c2_row.T
            ncross2 = jax.lax.dot_general(
                cbt * -2.0, z_blk, (((0,), (0,)), ((), ())),
                preferred_element_type=jnp.float32)
            d2 = ncross2 + c2

            idx = jnp.argmin(d2, axis=0, keepdims=True)

            onehot = (jax.lax.broadcasted_iota(jnp.int32, d2.shape, 0) == idx
                      ).astype(jnp.float32)
            z_q = jax.lax.dot_general(
                cbt, onehot, (((1,), (0,)), ((), ())),
                preferred_element_type=jnp.float32)

            diff = z_q - z_blk
            zq_ref[g, n * ds:(n + 1) * ds] = z_blk + diff
            part = jnp.sum(diff * diff)
            loss = part if loss is None else loss + part
            idx_rows.append(idx.astype(jnp.int32))

        idx_mat = jnp.concatenate(idx_rows, axis=0)
        idx_ref[g * hw:(g + 1) * hw, :] = idx_mat.T
    loss_ref[0, 0, 0] = loss


def kernel(z, codebooks):
    B, D, H, W = z.shape
    N, M, ds = codebooks.shape
    HW = H * W
    T = B * HW

    zr = z.reshape(B, D, HW)
    cbt = codebooks.transpose(0, 2, 1)

    BG = 2
    zq3, idx2, loss_p = pl.pallas_call(
        _vq_batch_kernel,
        grid=(B // BG,),
        in_specs=[
            pl.BlockSpec((BG, D, HW), lambda b: (b, 0, 0)),
            pl.BlockSpec((N, ds, M), lambda b: (0, 0, 0)),
        ],
        out_specs=[
            pl.BlockSpec((BG, D, HW), lambda b: (b, 0, 0)),
            pl.BlockSpec((BG * HW, N), lambda b: (b, 0)),
            pl.BlockSpec((1, 1, 1), lambda b: (b, 0, 0),
                         memory_space=pltpu.SMEM),
        ],
        out_shape=[
            jax.ShapeDtypeStruct((B, D, HW), jnp.float32),
            jax.ShapeDtypeStruct((T, N), jnp.int32),
            jax.ShapeDtypeStruct((B // BG, 1, 1), jnp.float32),
        ],
        compiler_params=pltpu.CompilerParams(
            dimension_semantics=("parallel",)),
    )(zr, cbt)

    z_q_out = zq3.reshape(B, D, H, W)
    loss = jnp.sum(loss_p) / jnp.float32(N * T * ds)
    return (z_q_out, loss, loss, idx2)
